# Initial kernel scaffold; baseline (speedup 1.0000x reference)
import jax
import jax.numpy as jnp
from jax.experimental import pallas as pl


def kernel(card, seen_mask, flipped, flipped_valid, t, W):
    raise NotImplementedError("write your pallas kernel here")


# speedup vs baseline: 12.7439x; 12.7439x over previous
import jax
import jax.numpy as jnp
from jax.experimental import pallas as pl

B = 4096
TWO_N = 128
N = 64
OUT_W = TWO_N * N + N + 2
ROWS = 128


def _body(card_ref, seen_ref, flip_ref, valid_ref, t_ref, out_ref):
    R = out_ref.shape[0]
    lane = jax.lax.broadcasted_iota(jnp.int32, (R, 128), 1)
    mod64 = jnp.bitwise_and(lane, 63)
    hi = lane >= 64

    card = card_ref[...]
    seen = seen_ref[...]
    cardm = jnp.where(seen, card, 64)

    for i in range(N):
        c0 = cardm[:, 2 * i : 2 * i + 1]
        c1 = cardm[:, 2 * i + 1 : 2 * i + 2]
        csel = jnp.where(hi, c1, c0)
        out_ref[:, 128 * i : 128 * (i + 1)] = jnp.where(
            csel == mod64, 1.0, 0.0
        )

    f = flip_ref[...]
    fc = jnp.sum(jnp.where(lane == f, card, 0), axis=1, keepdims=True)
    valid = valid_ref[...]
    par = jnp.bitwise_and(t_ref[...], 1)
    flip_val = jnp.where(lane == fc, valid, 0.0)
    par_val = jnp.where((lane - 64) == par, 1.0, 0.0)
    tail = jnp.where(lane < 64, flip_val, par_val)
    out_ref[:, TWO_N * N : OUT_W] = tail[:, : N + 2]


def kernel(card, seen_mask, flipped, flipped_valid, t, W):
    del W
    card = card.astype(jnp.int32)
    seen = seen_mask
    flip = flipped.astype(jnp.int32).reshape(B, 1)
    valid = flipped_valid.astype(jnp.float32).reshape(B, 1)
    t32 = t.astype(jnp.int32).reshape(B, 1)

    grid = (B // ROWS,)
    out = pl.pallas_call(
        _body,
        grid=grid,
        in_specs=[
            pl.BlockSpec((ROWS, TWO_N), lambda i: (i, 0)),
            pl.BlockSpec((ROWS, TWO_N), lambda i: (i, 0)),
            pl.BlockSpec((ROWS, 1), lambda i: (i, 0)),
            pl.BlockSpec((ROWS, 1), lambda i: (i, 0)),
            pl.BlockSpec((ROWS, 1), lambda i: (i, 0)),
        ],
        out_specs=pl.BlockSpec((ROWS, OUT_W), lambda i: (i, 0)),
        out_shape=jax.ShapeDtypeStruct((B, OUT_W), jnp.float32),
    )(card, seen, flip, valid, t32)
    return out.reshape(B, 1, OUT_W)
